# baseline (speedup 1.0000x reference)
import functools
import math

import jax
import jax.numpy as jnp
import numpy as np
from jax import lax
from jax.experimental import pallas as pl
from jax.experimental.pallas import tpu as pltpu
from jax.experimental.pallas import tpu_sc as plsc

_D = 128
_SEQ = 200
_CHUNK = 128
_NC, _NS = 2, 16
_NW = _NC * _NS
_PPC = _CHUNK // 32


def _pe_np(d_model: int, seq: int) -> np.ndarray:
    pos = np.arange(seq, dtype=np.float32)[:, None]
    div = np.exp(np.arange(0, d_model, 2, dtype=np.float32)
                 * (-math.log(10000.0) / d_model))
    pe = np.zeros((seq, d_model), dtype=np.float32)
    pe[:, 0::2] = np.sin(pos * div)
    pe[:, 1::2] = np.cos(pos * div)
    return pe


_PE = _pe_np(_D, _SEQ)


def _oidx_np(n_per_w: int, n_seq_w: int, seq: int) -> np.ndarray:
    w = np.arange(_NW, dtype=np.int32)[:, None, None]
    p = np.arange(seq, dtype=np.int32)[None, :, None]
    b = np.arange(n_seq_w, dtype=np.int32)[None, None, :]
    full = w * n_per_w + b * seq + p
    return full.reshape(_NW, (seq * n_seq_w) // _CHUNK, _CHUNK)


def kernel(token_ids, table):
    B, L = token_ids.shape
    V, D = table.shape
    N = B * L
    n_per_w = N // _NW
    n_chunks = n_per_w // _CHUNK
    n_seq_w = n_per_w // _SEQ

    perm_ids = token_ids.reshape(_NW, n_seq_w, L).transpose(0, 2, 1).reshape(N)
    pe = jnp.asarray(_PE)
    oidx = jnp.asarray(_oidx_np(n_per_w, n_seq_w, L))

    mesh = plsc.VectorSubcoreMesh(core_axis_name="c", subcore_axis_name="s")

    @functools.partial(
        pl.kernel,
        mesh=mesh,
        out_type=jax.ShapeDtypeStruct((N, D), jnp.float32),
        scratch_types=[
            pltpu.VMEM((n_per_w,), jnp.int32),
            pltpu.VMEM((n_chunks, _CHUNK), jnp.int32),
            pltpu.VMEM((_CHUNK, _D), jnp.float32),
            pltpu.VMEM((_CHUNK, _D), jnp.float32),
            pltpu.VMEM((_CHUNK, _D), jnp.float32),
            pltpu.VMEM((_CHUNK, _D), jnp.float32),
            pltpu.VMEM((_SEQ, _D), jnp.float32),
            pltpu.SemaphoreType.DMA,
            pltpu.SemaphoreType.DMA,
            pltpu.SemaphoreType.DMA,
            pltpu.SemaphoreType.DMA,
        ],
    )
    def _emb(pidx_hbm, oidx_hbm, pe_hbm, table_hbm, out_hbm,
             pidx_v, oidx_v, rowsa, rowsb, oba, obb, pe_v,
             ga, gb, oa, ob):
        wid = lax.axis_index("s") * _NC + lax.axis_index("c")
        base = wid * n_per_w
        pltpu.sync_copy(pidx_hbm.at[pl.ds(base, n_per_w)], pidx_v)
        pltpu.sync_copy(oidx_hbm.at[wid], oidx_v)
        pltpu.sync_copy(pe_hbm, pe_v)

        rows = (rowsa, rowsb)
        obuf = (oba, obb)
        gsem = (ga, gb)
        osem = (oa, ob)

        def g_desc(c, b):
            return pltpu.make_async_copy(
                table_hbm.at[pidx_v.at[pl.ds(c * _CHUNK, _CHUNK)]],
                rows[b], gsem[b])

        def o_desc(c, b):
            return pltpu.make_async_copy(
                obuf[b], out_hbm.at[oidx_v.at[c]], osem[b])

        g_desc(0, 0).start()
        g_desc(1, 1).start()

        def outer(i, carry):
            c0 = i * 2
            for b in range(2):
                c = c0 + b
                g_desc(c, b).wait()

                @pl.when(c >= 2)
                def _():
                    o_desc(c - 2, b).wait()

                for q in range(_PPC):
                    p = c * _PPC + q
                    pe_regs = [pe_v[p, pl.ds(k * 16, 16)] for k in range(8)]

                    def _rows(r, _q=q, _pe=pe_regs):
                        row = _q * 32 + r
                        for k in range(8):
                            sl = pl.ds(k * 16, 16)
                            obuf[b][row, sl] = rows[b][row, sl] + _pe[k]

                    plsc.parallel_loop(0, 32, unroll=4)(_rows)

                @pl.when(c + 2 < n_chunks)
                def _():
                    g_desc(c + 2, b).start()

                o_desc(c, b).start()
            return carry

        lax.fori_loop(0, n_chunks // 2, outer, 0)
        o_desc(n_chunks - 2, 0).wait()
        o_desc(n_chunks - 1, 1).wait()

    out = _emb(perm_ids, oidx, pe, table)
    return out.reshape(B, L, D)


# speedup vs baseline: 1.0571x; 1.0571x over previous
import functools
import math

import jax
import jax.numpy as jnp
import numpy as np
from jax import lax
from jax.experimental import pallas as pl
from jax.experimental.pallas import tpu as pltpu
from jax.experimental.pallas import tpu_sc as plsc

_D = 128
_SEQ = 200
_NC, _NS = 2, 16
_NW = _NC * _NS
_GP, _GS = 8, 4
_CHUNK = 256
_NBUF = 3


def _pe_np(d_model: int, seq: int) -> np.ndarray:
    pos = np.arange(seq, dtype=np.float32)[:, None]
    div = np.exp(np.arange(0, d_model, 2, dtype=np.float32)
                 * (-math.log(10000.0) / d_model))
    pe = np.zeros((seq, d_model), dtype=np.float32)
    pe[:, 0::2] = np.sin(pos * div)
    pe[:, 1::2] = np.cos(pos * div)
    return pe


_PE = _pe_np(_D, _SEQ)


def _oidx_np(n_pos_w: int, n_seq_w: int, seq: int) -> np.ndarray:
    g = np.arange(_GP, dtype=np.int32)[:, None, None, None]
    h = np.arange(_GS, dtype=np.int32)[None, :, None, None]
    c = np.arange(n_pos_w, dtype=np.int32)[None, None, :, None]
    r = np.arange(n_seq_w, dtype=np.int32)[None, None, None, :]
    full = (h * n_seq_w + r) * seq + (g * n_pos_w + c)
    return full.reshape(_NW, n_pos_w, _CHUNK // 128, 128)


def kernel(token_ids, table):
    B, L = token_ids.shape
    V, D = table.shape
    N = B * L
    n_pos_w = L // _GP
    n_seq_w = B // _GS
    n_chunks = n_pos_w

    perm_ids = (token_ids.reshape(_GS, n_seq_w, _GP, n_pos_w)
                .transpose(2, 0, 3, 1)
                .reshape(_NW, 2 * n_chunks, 128))
    pe = jnp.asarray(_PE.reshape(_GP, n_pos_w, _D))
    oidx = jnp.asarray(_oidx_np(n_pos_w, n_seq_w, L))

    mesh = plsc.VectorSubcoreMesh(core_axis_name="c", subcore_axis_name="s")

    @functools.partial(
        pl.kernel,
        mesh=mesh,
        out_type=jax.ShapeDtypeStruct((N, D), jnp.float32),
        scratch_types=[
            pltpu.VMEM((2 * n_chunks, 128), jnp.int32),
            pltpu.VMEM((n_chunks, 2, 128), jnp.int32),
            pltpu.VMEM((_CHUNK, _D), jnp.float32),
            pltpu.VMEM((_CHUNK, _D), jnp.float32),
            pltpu.VMEM((_CHUNK, _D), jnp.float32),
            pltpu.VMEM((n_pos_w, _D), jnp.float32),
            pltpu.SemaphoreType.DMA,
            pltpu.SemaphoreType.DMA,
            pltpu.SemaphoreType.DMA,
            pltpu.SemaphoreType.DMA,
            pltpu.SemaphoreType.DMA,
            pltpu.SemaphoreType.DMA,
        ],
    )
    def _emb(pidx_hbm, oidx_hbm, pe_hbm, table_hbm, out_hbm,
             pidx_v, oidx_v, r0, r1, r2, pe_v,
             g0, g1, g2, o0, o1, o2):
        wid = lax.axis_index("s") * _NC + lax.axis_index("c")
        pltpu.sync_copy(pidx_hbm.at[wid], pidx_v)
        pltpu.sync_copy(oidx_hbm.at[wid], oidx_v)
        pltpu.sync_copy(pe_hbm.at[wid // _GS], pe_v)

        rows = (r0, r1, r2)
        gsem = (g0, g1, g2)
        osem = (o0, o1, o2)

        def g_descs(c, b):
            return [pltpu.make_async_copy(
                        table_hbm.at[pidx_v.at[2 * c + s]],
                        rows[b].at[pl.ds(s * 128, 128)], gsem[b])
                    for s in range(2)]

        def o_descs(c, b):
            return [pltpu.make_async_copy(
                        rows[b].at[pl.ds(s * 128, 128)],
                        out_hbm.at[oidx_v.at[c, s]], osem[b])
                    for s in range(2)]

        def start(descs):
            for d in descs:
                d.start()

        def drain(descs):
            for d in descs:
                d.wait()

        for b in range(_NBUF):
            start(g_descs(b, b))

        def body(c, b, tail=False):
            drain(g_descs(c, b))

            pe_regs = [pe_v[c, pl.ds(k * 16, 16)] for k in range(8)]

            def _rows(r, _pe=pe_regs, _buf=rows[b]):
                for k in range(8):
                    sl = pl.ds(k * 16, 16)
                    _buf[r, sl] = _buf[r, sl] + _pe[k]

            plsc.parallel_loop(0, _CHUNK, unroll=4)(_rows)

            start(o_descs(c, b))

            if not tail:
                @pl.when(jnp.logical_and(c >= 1, c + 2 < n_chunks))
                def _():
                    drain(o_descs(c - 1, (b + 2) % _NBUF))
                    start(g_descs(c + 2, (b + 2) % _NBUF))

        def outer(i, carry):
            for b in range(_NBUF):
                body(i * _NBUF + b, b)
            return carry

        lax.fori_loop(0, (n_chunks - 1) // _NBUF, outer, 0)
        body(n_chunks - 1, (n_chunks - 1) % _NBUF, tail=True)
        for c in range(n_chunks - 3, n_chunks):
            drain(o_descs(c, c % _NBUF))

    out = _emb(perm_ids, oidx, pe, table)
    return out.reshape(B, L, D)
